# Initial kernel scaffold; baseline (speedup 1.0000x reference)
import functools

import jax
import jax.numpy as jnp
from jax import lax
from jax.experimental import pallas as pl
from jax.experimental.pallas import tpu as pltpu
from jax.experimental.pallas import tpu_sc as plsc

N = 10000
E = 320000
DF = 128
H = 32
LAT = 16

NC = 2
NS = 16
NW = NC * NS
CH = 125
CHUNKS = E // CH
TCH = CHUNKS // NW
NB = 4
CP_TILES = 10
ROWS_PT = N // CP_TILES

_mesh = plsc.VectorSubcoreMesh(core_axis_name="c", subcore_axis_name="s")


@functools.partial(
    pl.kernel,
    out_type=jax.ShapeDtypeStruct((N, 16), jnp.float32),
    mesh=_mesh,
    scratch_types=[
        pltpu.VMEM_SHARED((N, 8), jnp.float32),
        pltpu.VMEM((TCH, CH), jnp.int32),
        pltpu.VMEM((CH, 8), jnp.float32),
        pltpu.SemaphoreType.DMA,
    ],
)
def _deg_kernel(dst2, ones, zeros, out, deg_sh, idx_d, obuf, sem):
    cid = lax.axis_index("c")
    sid = lax.axis_index("s")
    wid = sid * NC + cid

    @pl.when(sid < CP_TILES)
    def _():
        r0 = sid * ROWS_PT
        pltpu.sync_copy(zeros.at[pl.ds(r0, ROWS_PT), pl.ds(0, 8)],
                        deg_sh.at[pl.ds(r0, ROWS_PT)])

    pltpu.sync_copy(dst2.at[pl.ds(wid * TCH, TCH)], idx_d)
    pltpu.sync_copy(ones, obuf)
    plsc.subcore_barrier()

    def step(j0, carry):
        descs = []
        for b in range(8):
            j = j0 * 8 + b
            descs.append(
                pltpu.async_copy(obuf, deg_sh.at[idx_d.at[j]], sem, add=True))
        for d in descs:
            d.wait()
        return carry

    lax.fori_loop(0, TCH // 8, step, 0)
    plsc.subcore_barrier()

    @pl.when(sid < CP_TILES)
    def _():
        r0 = sid * ROWS_PT
        pltpu.sync_copy(deg_sh.at[pl.ds(r0, ROWS_PT)],
                        out.at[pl.ds(r0, ROWS_PT), pl.ds(cid * 8, 8)])


@functools.partial(
    pl.kernel,
    out_type=jax.ShapeDtypeStruct((NC, N, H), jnp.float32),
    mesh=_mesh,
    scratch_types=[
        pltpu.VMEM_SHARED((N, H), jnp.float32),
        pltpu.VMEM((TCH, CH), jnp.int32),
        pltpu.VMEM((TCH, CH), jnp.int32),
        pltpu.VMEM((NB, CH, H), jnp.float32),
    ] + [pltpu.SemaphoreType.DMA] * NB,
)
def _prop_kernel(table, src2, dst2, zeros, out, acc_sh, idx_s, idx_d, rows,
                 *gsems):
    cid = lax.axis_index("c")
    sid = lax.axis_index("s")
    wid = sid * NC + cid

    @pl.when(sid < CP_TILES)
    def _():
        r0 = sid * ROWS_PT
        pltpu.sync_copy(zeros.at[pl.ds(r0, ROWS_PT)],
                        acc_sh.at[pl.ds(r0, ROWS_PT)])

    base = wid * TCH
    pltpu.sync_copy(src2.at[pl.ds(base, TCH)], idx_s)
    pltpu.sync_copy(dst2.at[pl.ds(base, TCH)], idx_d)
    plsc.subcore_barrier()

    for b in range(NB):
        pltpu.async_copy(table.at[idx_s.at[b]], rows.at[b], gsems[b])

    def step(j0, carry):
        for b in range(NB):
            j = j0 * NB + b
            pltpu.make_async_copy(table.at[idx_s.at[j]], rows.at[b],
                                  gsems[b]).wait()
            pltpu.sync_copy(rows.at[b], acc_sh.at[idx_d.at[j]], add=True)
            pltpu.async_copy(table.at[idx_s.at[j + NB]], rows.at[b], gsems[b])
        return carry

    lax.fori_loop(0, TCH // NB - 1, step, 0)

    for b in range(NB):
        j = TCH - NB + b
        pltpu.make_async_copy(table.at[idx_s.at[j]], rows.at[b],
                              gsems[b]).wait()
        pltpu.sync_copy(rows.at[b], acc_sh.at[idx_d.at[j]], add=True)

    plsc.subcore_barrier()

    @pl.when(sid < CP_TILES)
    def _():
        r0 = sid * ROWS_PT
        pltpu.sync_copy(acc_sh.at[pl.ds(r0, ROWS_PT)],
                        out.at[cid, pl.ds(r0, ROWS_PT)])


_GRID = 10
_BR = N // _GRID


def _dis(degp_ref):
    deg = degp_ref[:, 0:1] + degp_ref[:, 8:9] + 1.0
    return lax.rsqrt(deg)


def _tc1_body(x_ref, w1_ref, degp_ref, g2_ref):
    g = jnp.dot(x_ref[...], w1_ref[...], preferred_element_type=jnp.float32)
    g2_ref[...] = g * _dis(degp_ref)


_tc1 = pl.pallas_call(
    _tc1_body,
    grid=(_GRID,),
    in_specs=[
        pl.BlockSpec((_BR, DF), lambda i: (i, 0)),
        pl.BlockSpec((DF, H), lambda i: (0, 0)),
        pl.BlockSpec((_BR, 16), lambda i: (i, 0)),
    ],
    out_specs=pl.BlockSpec((_BR, H), lambda i: (i, 0)),
    out_shape=jax.ShapeDtypeStruct((N, H), jnp.float32),
)


def _tc2_body(g2_ref, acc_ref, degp_ref, b1_ref, h2_ref):
    dis = _dis(degp_ref)
    s = dis * (g2_ref[...] + acc_ref[0] + acc_ref[1]) + b1_ref[...]
    h2_ref[...] = dis * jnp.maximum(s, 0.0)


_tc2 = pl.pallas_call(
    _tc2_body,
    grid=(_GRID,),
    in_specs=[
        pl.BlockSpec((_BR, H), lambda i: (i, 0)),
        pl.BlockSpec((NC, _BR, H), lambda i: (0, i, 0)),
        pl.BlockSpec((_BR, 16), lambda i: (i, 0)),
        pl.BlockSpec((1, H), lambda i: (0, 0)),
    ],
    out_specs=pl.BlockSpec((_BR, H), lambda i: (i, 0)),
    out_shape=jax.ShapeDtypeStruct((N, H), jnp.float32),
)


def _tc3_body(h2_ref, acc_ref, degp_ref, wm_ref, bm_ref, wv_ref, bv_ref,
              mean_ref, var_ref):
    dis = _dis(degp_ref)
    p = dis * (h2_ref[...] + acc_ref[0] + acc_ref[1])
    mean_ref[...] = jnp.dot(p, wm_ref[...],
                            preferred_element_type=jnp.float32) + bm_ref[...]
    var_ref[...] = jnp.dot(p, wv_ref[...],
                           preferred_element_type=jnp.float32) + bv_ref[...]


_tc3 = pl.pallas_call(
    _tc3_body,
    grid=(_GRID,),
    in_specs=[
        pl.BlockSpec((_BR, H), lambda i: (i, 0)),
        pl.BlockSpec((NC, _BR, H), lambda i: (0, i, 0)),
        pl.BlockSpec((_BR, 16), lambda i: (i, 0)),
        pl.BlockSpec((H, LAT), lambda i: (0, 0)),
        pl.BlockSpec((1, LAT), lambda i: (0, 0)),
        pl.BlockSpec((H, LAT), lambda i: (0, 0)),
        pl.BlockSpec((1, LAT), lambda i: (0, 0)),
    ],
    out_specs=[
        pl.BlockSpec((_BR, LAT), lambda i: (i, 0)),
        pl.BlockSpec((_BR, LAT), lambda i: (i, 0)),
    ],
    out_shape=[
        jax.ShapeDtypeStruct((N, LAT), jnp.float32),
        jax.ShapeDtypeStruct((N, LAT), jnp.float32),
    ],
)


@jax.jit
def kernel(features, edge_index, W1, b1, Wm, bm, Wv, bv):
    src2 = edge_index[0].reshape(CHUNKS, CH)
    dst2 = edge_index[1].reshape(CHUNKS, CH)
    zeros = jnp.zeros((N, H), jnp.float32)
    ones = jnp.ones((CH, 8), jnp.float32)

    degp = _deg_kernel(dst2, ones, zeros)
    g2 = _tc1(features, W1, degp)
    acc1 = _prop_kernel(g2, src2, dst2, zeros)
    h2 = _tc2(g2, acc1, degp, b1.reshape(1, H))
    acc2 = _prop_kernel(h2, src2, dst2, zeros)
    mean, var = _tc3(h2, acc2, degp, Wm, bm.reshape(1, LAT),
                     Wv, bv.reshape(1, LAT))
    return (mean, var)


# speedup vs baseline: 61.2474x; 61.2474x over previous
import functools

import jax
import jax.numpy as jnp
from jax import lax
from jax.experimental import pallas as pl
from jax.experimental.pallas import tpu as pltpu
from jax.experimental.pallas import tpu_sc as plsc

N = 10000
E = 320000
DF = 128
H = 32
LAT = 16

NC = 2
NS = 16
NW = NC * NS
CH = 125
CHUNKS = E // CH
TCH = CHUNKS // NW
NB = 4
CP_TILES = 10
ROWS_PT = N // CP_TILES

_mesh = plsc.VectorSubcoreMesh(core_axis_name="c", subcore_axis_name="s")


@functools.partial(
    pl.kernel,
    out_type=jax.ShapeDtypeStruct((NC, N, 8), jnp.float32),
    mesh=_mesh,
    compiler_params=pltpu.CompilerParams(use_tc_tiling_on_sc=False),
    scratch_types=[
        pltpu.VMEM_SHARED((N, 8), jnp.float32),
        pltpu.VMEM((TCH, CH), jnp.int32),
        pltpu.VMEM((CH, 8), jnp.float32),
        pltpu.SemaphoreType.DMA,
    ],
)
def _deg_kernel(dst2, ones, zeros8, out, deg_sh, idx_d, obuf, sem):
    cid = lax.axis_index("c")
    sid = lax.axis_index("s")
    wid = sid * NC + cid

    @pl.when(sid < CP_TILES)
    def _():
        r0 = sid * ROWS_PT
        pltpu.sync_copy(zeros8.at[pl.ds(r0, ROWS_PT)],
                        deg_sh.at[pl.ds(r0, ROWS_PT)])

    pltpu.sync_copy(dst2.at[pl.ds(wid * TCH, TCH)], idx_d)
    pltpu.sync_copy(ones, obuf)
    plsc.subcore_barrier()

    def step(j0, carry):
        descs = []
        for b in range(8):
            j = j0 * 8 + b
            descs.append(
                pltpu.async_copy(obuf, deg_sh.at[idx_d.at[j]], sem, add=True))
        for d in descs:
            d.wait()
        return carry

    lax.fori_loop(0, TCH // 8, step, 0)
    plsc.subcore_barrier()

    @pl.when(sid < CP_TILES)
    def _():
        r0 = sid * ROWS_PT
        pltpu.sync_copy(deg_sh.at[pl.ds(r0, ROWS_PT)],
                        out.at[cid, pl.ds(r0, ROWS_PT)])


@functools.partial(
    pl.kernel,
    out_type=jax.ShapeDtypeStruct((NC, N, H), jnp.float32),
    mesh=_mesh,
    compiler_params=pltpu.CompilerParams(use_tc_tiling_on_sc=False),
    scratch_types=[
        pltpu.VMEM_SHARED((N, H), jnp.float32),
        pltpu.VMEM((TCH, CH), jnp.int32),
        pltpu.VMEM((TCH, CH), jnp.int32),
        pltpu.VMEM((NB, CH, H), jnp.float32),
    ] + [pltpu.SemaphoreType.DMA] * NB,
)
def _prop_kernel(table, src2, dst2, zeros, out, acc_sh, idx_s, idx_d, rows,
                 *gsems):
    cid = lax.axis_index("c")
    sid = lax.axis_index("s")
    wid = sid * NC + cid

    @pl.when(sid < CP_TILES)
    def _():
        r0 = sid * ROWS_PT
        pltpu.sync_copy(zeros.at[pl.ds(r0, ROWS_PT)],
                        acc_sh.at[pl.ds(r0, ROWS_PT)])

    base = wid * TCH
    pltpu.sync_copy(src2.at[pl.ds(base, TCH)], idx_s)
    pltpu.sync_copy(dst2.at[pl.ds(base, TCH)], idx_d)
    plsc.subcore_barrier()

    for b in range(NB):
        pltpu.async_copy(table.at[idx_s.at[b]], rows.at[b], gsems[b])

    def step(j0, carry):
        for b in range(NB):
            j = j0 * NB + b
            pltpu.make_async_copy(table.at[idx_s.at[j]], rows.at[b],
                                  gsems[b]).wait()
            pltpu.sync_copy(rows.at[b], acc_sh.at[idx_d.at[j]], add=True)
            pltpu.async_copy(table.at[idx_s.at[j + NB]], rows.at[b], gsems[b])
        return carry

    lax.fori_loop(0, TCH // NB - 1, step, 0)

    for b in range(NB):
        j = TCH - NB + b
        pltpu.make_async_copy(table.at[idx_s.at[j]], rows.at[b],
                              gsems[b]).wait()
        pltpu.sync_copy(rows.at[b], acc_sh.at[idx_d.at[j]], add=True)

    plsc.subcore_barrier()

    @pl.when(sid < CP_TILES)
    def _():
        r0 = sid * ROWS_PT
        pltpu.sync_copy(acc_sh.at[pl.ds(r0, ROWS_PT)],
                        out.at[cid, pl.ds(r0, ROWS_PT)])


_GRID = 10
_BR = N // _GRID


def _dis(degp_ref):
    deg = degp_ref[0][:, 0:1] + degp_ref[1][:, 0:1] + 1.0
    return lax.rsqrt(deg)


def _tc1_body(x_ref, w1_ref, degp_ref, g2_ref):
    g = jnp.dot(x_ref[...], w1_ref[...], preferred_element_type=jnp.float32)
    g2_ref[...] = g * _dis(degp_ref)


_tc1 = pl.pallas_call(
    _tc1_body,
    grid=(_GRID,),
    in_specs=[
        pl.BlockSpec((_BR, DF), lambda i: (i, 0)),
        pl.BlockSpec((DF, H), lambda i: (0, 0)),
        pl.BlockSpec((NC, _BR, 8), lambda i: (0, i, 0)),
    ],
    out_specs=pl.BlockSpec((_BR, H), lambda i: (i, 0)),
    out_shape=jax.ShapeDtypeStruct((N, H), jnp.float32),
)


def _tc2_body(g2_ref, acc_ref, degp_ref, b1_ref, h2_ref):
    dis = _dis(degp_ref)
    s = dis * (g2_ref[...] + acc_ref[0] + acc_ref[1]) + b1_ref[...]
    h2_ref[...] = dis * jnp.maximum(s, 0.0)


_tc2 = pl.pallas_call(
    _tc2_body,
    grid=(_GRID,),
    in_specs=[
        pl.BlockSpec((_BR, H), lambda i: (i, 0)),
        pl.BlockSpec((NC, _BR, H), lambda i: (0, i, 0)),
        pl.BlockSpec((NC, _BR, 8), lambda i: (0, i, 0)),
        pl.BlockSpec((1, H), lambda i: (0, 0)),
    ],
    out_specs=pl.BlockSpec((_BR, H), lambda i: (i, 0)),
    out_shape=jax.ShapeDtypeStruct((N, H), jnp.float32),
)


def _tc3_body(h2_ref, acc_ref, degp_ref, wm_ref, bm_ref, wv_ref, bv_ref,
              mean_ref, var_ref):
    dis = _dis(degp_ref)
    p = dis * (h2_ref[...] + acc_ref[0] + acc_ref[1])
    mean_ref[...] = jnp.dot(p, wm_ref[...],
                            preferred_element_type=jnp.float32) + bm_ref[...]
    var_ref[...] = jnp.dot(p, wv_ref[...],
                           preferred_element_type=jnp.float32) + bv_ref[...]


_tc3 = pl.pallas_call(
    _tc3_body,
    grid=(_GRID,),
    in_specs=[
        pl.BlockSpec((_BR, H), lambda i: (i, 0)),
        pl.BlockSpec((NC, _BR, H), lambda i: (0, i, 0)),
        pl.BlockSpec((NC, _BR, 8), lambda i: (0, i, 0)),
        pl.BlockSpec((H, LAT), lambda i: (0, 0)),
        pl.BlockSpec((1, LAT), lambda i: (0, 0)),
        pl.BlockSpec((H, LAT), lambda i: (0, 0)),
        pl.BlockSpec((1, LAT), lambda i: (0, 0)),
    ],
    out_specs=[
        pl.BlockSpec((_BR, LAT), lambda i: (i, 0)),
        pl.BlockSpec((_BR, LAT), lambda i: (i, 0)),
    ],
    out_shape=[
        jax.ShapeDtypeStruct((N, LAT), jnp.float32),
        jax.ShapeDtypeStruct((N, LAT), jnp.float32),
    ],
)


@jax.jit
def kernel(features, edge_index, W1, b1, Wm, bm, Wv, bv):
    src2 = edge_index[0].reshape(CHUNKS, CH)
    dst2 = edge_index[1].reshape(CHUNKS, CH)
    zeros = jnp.zeros((N, H), jnp.float32)
    zeros8 = jnp.zeros((N, 8), jnp.float32)
    ones = jnp.ones((CH, 8), jnp.float32)

    degp = _deg_kernel(dst2, ones, zeros8)
    g2 = _tc1(features, W1, degp)
    acc1 = _prop_kernel(g2, src2, dst2, zeros)
    h2 = _tc2(g2, acc1, degp, b1.reshape(1, H))
    acc2 = _prop_kernel(h2, src2, dst2, zeros)
    mean, var = _tc3(h2, acc2, degp, Wm, bm.reshape(1, LAT),
                     Wv, bv.reshape(1, LAT))
    return (mean, var)


# speedup vs baseline: 63.4376x; 1.0358x over previous
import functools

import jax
import jax.numpy as jnp
from jax import lax
from jax.experimental import pallas as pl
from jax.experimental.pallas import tpu as pltpu
from jax.experimental.pallas import tpu_sc as plsc

N = 10000
E = 320000
DF = 128
H = 32
LAT = 16

NC = 2
NS = 16
NW = NC * NS
CH = 125
CHUNKS = E // CH
TCH = CHUNKS // NW
NB = 8
CP_TILES = 10
ROWS_PT = N // CP_TILES

_mesh = plsc.VectorSubcoreMesh(core_axis_name="c", subcore_axis_name="s")


@functools.partial(
    pl.kernel,
    out_type=jax.ShapeDtypeStruct((NC, N, 8), jnp.float32),
    mesh=_mesh,
    compiler_params=pltpu.CompilerParams(use_tc_tiling_on_sc=False),
    scratch_types=[
        pltpu.VMEM_SHARED((N, 8), jnp.float32),
        pltpu.VMEM((TCH, CH), jnp.int32),
        pltpu.VMEM((CH, 8), jnp.float32),
        pltpu.SemaphoreType.DMA,
    ],
)
def _deg_kernel(dst2, ones, zeros8, out, deg_sh, idx_d, obuf, sem):
    cid = lax.axis_index("c")
    sid = lax.axis_index("s")
    wid = sid * NC + cid

    @pl.when(sid < CP_TILES)
    def _():
        r0 = sid * ROWS_PT
        pltpu.sync_copy(zeros8.at[pl.ds(r0, ROWS_PT)],
                        deg_sh.at[pl.ds(r0, ROWS_PT)])

    pltpu.sync_copy(dst2.at[pl.ds(wid * TCH, TCH)], idx_d)
    pltpu.sync_copy(ones, obuf)
    plsc.subcore_barrier()

    def step(j0, carry):
        descs = []
        for b in range(8):
            j = j0 * 8 + b
            descs.append(
                pltpu.async_copy(obuf, deg_sh.at[idx_d.at[j]], sem, add=True))
        for d in descs:
            d.wait()
        return carry

    lax.fori_loop(0, TCH // 8, step, 0)
    plsc.subcore_barrier()

    @pl.when(sid < CP_TILES)
    def _():
        r0 = sid * ROWS_PT
        pltpu.sync_copy(deg_sh.at[pl.ds(r0, ROWS_PT)],
                        out.at[cid, pl.ds(r0, ROWS_PT)])


@functools.partial(
    pl.kernel,
    out_type=jax.ShapeDtypeStruct((NC, N, H), jnp.float32),
    mesh=_mesh,
    compiler_params=pltpu.CompilerParams(use_tc_tiling_on_sc=False),
    scratch_types=[
        pltpu.VMEM_SHARED((N, H), jnp.float32),
        pltpu.VMEM((TCH, CH), jnp.int32),
        pltpu.VMEM((TCH, CH), jnp.int32),
        pltpu.VMEM((NB, CH, H), jnp.float32),
    ] + [pltpu.SemaphoreType.DMA] * NB,
)
def _prop_kernel(table, src2, dst2, zeros, out, acc_sh, idx_s, idx_d, rows,
                 *gsems):
    cid = lax.axis_index("c")
    sid = lax.axis_index("s")
    wid = sid * NC + cid

    @pl.when(sid < CP_TILES)
    def _():
        r0 = sid * ROWS_PT
        pltpu.sync_copy(zeros.at[pl.ds(r0, ROWS_PT)],
                        acc_sh.at[pl.ds(r0, ROWS_PT)])

    base = wid * TCH
    pltpu.sync_copy(src2.at[pl.ds(base, TCH)], idx_s)
    pltpu.sync_copy(dst2.at[pl.ds(base, TCH)], idx_d)
    plsc.subcore_barrier()

    for b in range(NB):
        pltpu.async_copy(table.at[idx_s.at[b]], rows.at[b], gsems[b])

    def step(j0, carry):
        for b in range(NB):
            j = j0 * NB + b
            pltpu.make_async_copy(table.at[idx_s.at[j]], rows.at[b],
                                  gsems[b]).wait()
            pltpu.sync_copy(rows.at[b], acc_sh.at[idx_d.at[j]], add=True)
            pltpu.async_copy(table.at[idx_s.at[j + NB]], rows.at[b], gsems[b])
        return carry

    lax.fori_loop(0, TCH // NB - 1, step, 0)

    for b in range(NB):
        j = TCH - NB + b
        pltpu.make_async_copy(table.at[idx_s.at[j]], rows.at[b],
                              gsems[b]).wait()
        pltpu.sync_copy(rows.at[b], acc_sh.at[idx_d.at[j]], add=True)

    plsc.subcore_barrier()

    @pl.when(sid < CP_TILES)
    def _():
        r0 = sid * ROWS_PT
        pltpu.sync_copy(acc_sh.at[pl.ds(r0, ROWS_PT)],
                        out.at[cid, pl.ds(r0, ROWS_PT)])


_GRID = 10
_BR = N // _GRID


def _dis(degp_ref):
    deg = degp_ref[0][:, 0:1] + degp_ref[1][:, 0:1] + 1.0
    return lax.rsqrt(deg)


def _tc1_body(x_ref, w1_ref, degp_ref, g2_ref):
    g = jnp.dot(x_ref[...], w1_ref[...], preferred_element_type=jnp.float32)
    g2_ref[...] = g * _dis(degp_ref)


_tc1 = pl.pallas_call(
    _tc1_body,
    grid=(_GRID,),
    in_specs=[
        pl.BlockSpec((_BR, DF), lambda i: (i, 0)),
        pl.BlockSpec((DF, H), lambda i: (0, 0)),
        pl.BlockSpec((NC, _BR, 8), lambda i: (0, i, 0)),
    ],
    out_specs=pl.BlockSpec((_BR, H), lambda i: (i, 0)),
    out_shape=jax.ShapeDtypeStruct((N, H), jnp.float32),
)


def _tc2_body(g2_ref, acc_ref, degp_ref, b1_ref, h2_ref):
    dis = _dis(degp_ref)
    s = dis * (g2_ref[...] + acc_ref[0] + acc_ref[1]) + b1_ref[...]
    h2_ref[...] = dis * jnp.maximum(s, 0.0)


_tc2 = pl.pallas_call(
    _tc2_body,
    grid=(_GRID,),
    in_specs=[
        pl.BlockSpec((_BR, H), lambda i: (i, 0)),
        pl.BlockSpec((NC, _BR, H), lambda i: (0, i, 0)),
        pl.BlockSpec((NC, _BR, 8), lambda i: (0, i, 0)),
        pl.BlockSpec((1, H), lambda i: (0, 0)),
    ],
    out_specs=pl.BlockSpec((_BR, H), lambda i: (i, 0)),
    out_shape=jax.ShapeDtypeStruct((N, H), jnp.float32),
)


def _tc3_body(h2_ref, acc_ref, degp_ref, wm_ref, bm_ref, wv_ref, bv_ref,
              mean_ref, var_ref):
    dis = _dis(degp_ref)
    p = dis * (h2_ref[...] + acc_ref[0] + acc_ref[1])
    mean_ref[...] = jnp.dot(p, wm_ref[...],
                            preferred_element_type=jnp.float32) + bm_ref[...]
    var_ref[...] = jnp.dot(p, wv_ref[...],
                           preferred_element_type=jnp.float32) + bv_ref[...]


_tc3 = pl.pallas_call(
    _tc3_body,
    grid=(_GRID,),
    in_specs=[
        pl.BlockSpec((_BR, H), lambda i: (i, 0)),
        pl.BlockSpec((NC, _BR, H), lambda i: (0, i, 0)),
        pl.BlockSpec((NC, _BR, 8), lambda i: (0, i, 0)),
        pl.BlockSpec((H, LAT), lambda i: (0, 0)),
        pl.BlockSpec((1, LAT), lambda i: (0, 0)),
        pl.BlockSpec((H, LAT), lambda i: (0, 0)),
        pl.BlockSpec((1, LAT), lambda i: (0, 0)),
    ],
    out_specs=[
        pl.BlockSpec((_BR, LAT), lambda i: (i, 0)),
        pl.BlockSpec((_BR, LAT), lambda i: (i, 0)),
    ],
    out_shape=[
        jax.ShapeDtypeStruct((N, LAT), jnp.float32),
        jax.ShapeDtypeStruct((N, LAT), jnp.float32),
    ],
)


@jax.jit
def kernel(features, edge_index, W1, b1, Wm, bm, Wv, bv):
    src2 = edge_index[0].reshape(CHUNKS, CH)
    dst2 = edge_index[1].reshape(CHUNKS, CH)
    zeros = jnp.zeros((N, H), jnp.float32)
    zeros8 = jnp.zeros((N, 8), jnp.float32)
    ones = jnp.ones((CH, 8), jnp.float32)

    degp = _deg_kernel(dst2, ones, zeros8)
    g2 = _tc1(features, W1, degp)
    acc1 = _prop_kernel(g2, src2, dst2, zeros)
    h2 = _tc2(g2, acc1, degp, b1.reshape(1, H))
    acc2 = _prop_kernel(h2, src2, dst2, zeros)
    mean, var = _tc3(h2, acc2, degp, Wm, bm.reshape(1, LAT),
                     Wv, bv.reshape(1, LAT))
    return (mean, var)
